# baseline (speedup 1.0000x reference)
import functools

import jax
import jax.numpy as jnp
from jax import lax
from jax.experimental import pallas as pl
from jax.experimental.pallas import tpu as pltpu
from jax.experimental.pallas import tpu_sc as plsc

NUM_EMB = 1_000_000
FEAT = 64
PAD_FEAT = 128
B_TOTAL = 4096 * 200
NUM_WORKERS = 32
B_PER_W = B_TOTAL // NUM_WORKERS
CHUNK = 320
N_CHUNKS = B_PER_W // CHUNK
N_PAIRS = N_CHUNKS // 2


def _make_kernel():
    mesh = plsc.VectorSubcoreMesh(core_axis_name="c", subcore_axis_name="s")

    @functools.partial(
        pl.kernel,
        mesh=mesh,
        out_type=jax.ShapeDtypeStruct((B_TOTAL, PAD_FEAT), jnp.float32),
        scratch_types=[
            pltpu.VMEM((B_PER_W,), jnp.int32),
            pltpu.VMEM((CHUNK, PAD_FEAT), jnp.float32),
            pltpu.VMEM((CHUNK, PAD_FEAT), jnp.float32),
            pltpu.SemaphoreType.DMA,
            pltpu.SemaphoreType.DMA,
            pltpu.SemaphoreType.DMA,
            pltpu.SemaphoreType.DMA,
        ],
        compiler_params=pltpu.CompilerParams(use_tc_tiling_on_sc=False),
    )
    def gather_kernel(idx_hbm, table_hbm, out_hbm,
                      idx_v, rows0, rows1, sg0, sg1, so0, so1):
        wid = lax.axis_index("s") * 2 + lax.axis_index("c")
        base = wid * B_PER_W

        pltpu.sync_copy(idx_hbm.at[pl.ds(base, B_PER_W)], idx_v)

        def gather_start(c, rows, sem):
            pltpu.async_copy(
                table_hbm.at[idx_v.at[pl.ds(c * CHUNK, CHUNK)]], rows, sem)

        def gather_wait(c, rows, sem):
            pltpu.make_async_copy(
                table_hbm.at[idx_v.at[pl.ds(c * CHUNK, CHUNK)]], rows,
                sem).wait()

        def out_start(c, rows, sem):
            pltpu.async_copy(
                rows.at[:, pl.ds(0, FEAT)],
                out_hbm.at[pl.ds(base + c * CHUNK, CHUNK), pl.ds(0, FEAT)],
                sem)

        def out_wait(c, rows, sem):
            pltpu.make_async_copy(
                rows.at[:, pl.ds(0, FEAT)],
                out_hbm.at[pl.ds(base + c * CHUNK, CHUNK), pl.ds(0, FEAT)],
                sem).wait()

        gather_start(0, rows0, sg0)
        gather_start(1, rows1, sg1)
        gather_wait(0, rows0, sg0)
        out_start(0, rows0, so0)
        gather_wait(1, rows1, sg1)
        out_start(1, rows1, so1)

        @pl.loop(1, N_PAIRS)
        def _(g):
            c0 = g * 2
            out_wait(c0 - 2, rows0, so0)
            gather_start(c0, rows0, sg0)
            out_wait(c0 - 1, rows1, so1)
            gather_start(c0 + 1, rows1, sg1)
            gather_wait(c0, rows0, sg0)
            out_start(c0, rows0, so0)
            gather_wait(c0 + 1, rows1, sg1)
            out_start(c0 + 1, rows1, so1)

        out_wait(N_CHUNKS - 2, rows0, so0)
        out_wait(N_CHUNKS - 1, rows1, so1)

    return gather_kernel


_gather = _make_kernel()


def kernel(x, embedding):
    table = jnp.pad(embedding, ((0, 0), (0, PAD_FEAT - FEAT)))
    out = _gather(x.reshape(-1), table)
    out = out.reshape(x.shape[0], x.shape[1], PAD_FEAT)
    return lax.slice(out, (0, 0, 0), (x.shape[0], x.shape[1], FEAT))


# speedup vs baseline: 1.0959x; 1.0959x over previous
import functools

import jax
import jax.numpy as jnp
from jax import lax
from jax.experimental import pallas as pl
from jax.experimental.pallas import tpu as pltpu
from jax.experimental.pallas import tpu_sc as plsc

NUM_EMB = 1_000_000
FEAT = 64
PAD_FEAT = 128
B_TOTAL = 4096 * 200
NUM_WORKERS = 32
B_PER_W = B_TOTAL // NUM_WORKERS
CHUNK = 640
N_CHUNKS = B_PER_W // CHUNK
N_PAIRS = N_CHUNKS // 2


def _make_kernel():
    mesh = plsc.VectorSubcoreMesh(core_axis_name="c", subcore_axis_name="s")

    @functools.partial(
        pl.kernel,
        mesh=mesh,
        out_type=jax.ShapeDtypeStruct((B_TOTAL, PAD_FEAT), jnp.float32),
        scratch_types=[
            pltpu.VMEM((B_PER_W,), jnp.int32),
            pltpu.VMEM((CHUNK, FEAT), jnp.float32),
            pltpu.VMEM((CHUNK, FEAT), jnp.float32),
            pltpu.SemaphoreType.DMA,
            pltpu.SemaphoreType.DMA,
            pltpu.SemaphoreType.DMA,
            pltpu.SemaphoreType.DMA,
        ],
        compiler_params=pltpu.CompilerParams(use_tc_tiling_on_sc=False),
    )
    def gather_kernel(idx_hbm, table_hbm, out_hbm,
                      idx_v, rows0, rows1, sg0, sg1, so0, so1):
        wid = lax.axis_index("s") * 2 + lax.axis_index("c")
        base = wid * B_PER_W

        table = table_hbm

        pltpu.sync_copy(idx_hbm.at[pl.ds(base, B_PER_W)], idx_v)

        def gather_start(c, rows, sem):
            pltpu.async_copy(
                table.at[idx_v.at[pl.ds(c * CHUNK, CHUNK)]], rows, sem)

        def gather_wait(c, rows, sem):
            pltpu.make_async_copy(
                table.at[idx_v.at[pl.ds(c * CHUNK, CHUNK)]], rows,
                sem).wait()

        def out_start(c, rows, sem):
            pltpu.async_copy(
                rows,
                out_hbm.at[pl.ds(base + c * CHUNK, CHUNK), pl.ds(0, FEAT)],
                sem)

        def out_wait(c, rows, sem):
            pltpu.make_async_copy(
                rows,
                out_hbm.at[pl.ds(base + c * CHUNK, CHUNK), pl.ds(0, FEAT)],
                sem).wait()

        gather_start(0, rows0, sg0)
        gather_start(1, rows1, sg1)
        gather_wait(0, rows0, sg0)
        out_start(0, rows0, so0)
        gather_wait(1, rows1, sg1)
        out_start(1, rows1, so1)

        @pl.loop(1, N_PAIRS)
        def _(g):
            c0 = g * 2
            out_wait(c0 - 2, rows0, so0)
            gather_start(c0, rows0, sg0)
            out_wait(c0 - 1, rows1, so1)
            gather_start(c0 + 1, rows1, sg1)
            gather_wait(c0, rows0, sg0)
            out_start(c0, rows0, so0)
            gather_wait(c0 + 1, rows1, sg1)
            out_start(c0 + 1, rows1, so1)

        out_wait(N_CHUNKS - 2, rows0, so0)
        out_wait(N_CHUNKS - 1, rows1, so1)

    return gather_kernel


_gather = _make_kernel()


def kernel(x, embedding):
    out = _gather(x.reshape(-1), embedding)
    out = out.reshape(x.shape[0], x.shape[1], PAD_FEAT)
    return lax.slice(out, (0, 0, 0), (x.shape[0], x.shape[1], FEAT))
